# baseline (speedup 1.0000x reference)
import functools

import jax
import jax.numpy as jnp
from jax import lax
from jax.experimental import pallas as pl
from jax.experimental.pallas import tpu as pltpu
from jax.experimental.pallas import tpu_sc as plsc

NC = 2
NS = 16
CHUNK = 64


def _matmul_body(x_ref, w_ref, cj_ref, o_ref):
    o_ref[...] = (
        jnp.dot(x_ref[...], w_ref[...], preferred_element_type=jnp.float32)
        * cj_ref[...]
    )


def _combine_body(p_ref, ci_ref, o_ref):
    o_ref[...] = (p_ref[0] + p_ref[1]) * ci_ref[...]


def _make_sc_kernel(n, n_acc, d, n_chunks):
    del n
    rows_per_tile = n_acc // NS
    mesh = plsc.VectorSubcoreMesh(
        core_axis_name="c", subcore_axis_name="s",
        num_cores=NC, num_subcores=NS)

    @functools.partial(
        pl.kernel,
        mesh=mesh,
        out_type=jax.ShapeDtypeStruct((NC, n_acc, d), jnp.float32),
        scratch_types=[
            pltpu.VMEM((n_chunks * CHUNK,), jnp.int32),
            pltpu.VMEM((n_chunks // 2, 2 * CHUNK), jnp.int32),
            pltpu.VMEM((CHUNK, d), jnp.float32),
            pltpu.VMEM((CHUNK, d), jnp.float32),
            pltpu.VMEM((CHUNK, d), jnp.float32),
            pltpu.VMEM_SHARED((n_acc, d), jnp.float32),
            pltpu.SemaphoreType.DMA,
            pltpu.SemaphoreType.DMA,
            pltpu.SemaphoreType.DMA,
            pltpu.SemaphoreType.DMA,
            pltpu.SemaphoreType.DMA,
            pltpu.SemaphoreType.DMA,
        ],
    )
    def sc_kernel(h_hbm, src_hbm, dst_hbm, out_hbm,
                  src_v, dst_v, b0, b1, b2, acc,
                  g0, g1, g2, s0, s1, s2):
        c = lax.axis_index("c")
        s = lax.axis_index("s")
        wid = c * NS + s
        bufs = [b0, b1, b2]
        gsems = [g0, g1, g2]
        ssems = [s0, s1, s2]

        def gather(j, b):
            idx = src_v.at[pl.ds(j * CHUNK, CHUNK)]
            return pltpu.make_async_copy(h_hbm.at[idx], bufs[b % 3],
                                         gsems[b % 3])

        def scatter(row, off, b):
            idx = dst_v.at[row, pl.ds(off * CHUNK, CHUNK)]
            return pltpu.make_async_copy(bufs[b % 3], acc.at[idx],
                                         ssems[b % 3])

        pltpu.sync_copy(src_hbm.at[wid], src_v)
        pltpu.sync_copy(dst_hbm.at[wid], dst_v)

        zv = jnp.zeros((16,), jnp.float32)

        def zero_row(i, carry):
            for jj in range(d // 16):
                b0[i, pl.ds(jj * 16, 16)] = zv
            return carry

        lax.fori_loop(0, CHUNK, zero_row, 0)
        for k in range(rows_per_tile // CHUNK):
            pltpu.sync_copy(b0, acc.at[pl.ds(s * rows_per_tile + k * CHUNK, CHUNK)])
        plsc.subcore_barrier()


        def step(k, kr, p):
            kp = k if isinstance(k, int) else p
            gather(k, kp).wait()
            scatter(kr, kp % 2, kp).start(add=True)
            if kp % 6 != 0 or not isinstance(k, int) or k > 0:
                km = kp - 1
                scatter(kr - 1 if kp % 2 == 0 else kr, km % 2, km).wait()
            nk = k + 2
            if isinstance(k, int) and nk >= n_chunks:
                return
            gather(nk, kp + 2).start()

        gather(0, 0).start()
        gather(1, 1).start()
        for k in range(6):
            step(k, k // 2, k)

        @pl.loop(6, n_chunks - 6, step=6)
        def _group(j):
            jr = j // 2
            for p in range(6):
                step(j + p, jr + p // 2, p)

        kt = n_chunks - 6
        ktr = kt // 2
        for p in range(6):
            step(kt + p, ktr + p // 2, p)
        scatter(ktr + 2, 1, kt + 5).wait()
        plsc.subcore_barrier()

        base = s * rows_per_tile
        pltpu.sync_copy(acc.at[pl.ds(base, rows_per_tile)],
                        out_hbm.at[c, pl.ds(base, rows_per_tile)])

    return sc_kernel


def kernel(x, edge_index, cj, ci, W):
    n, d_in = x.shape
    d_out = W.shape[1]
    e = edge_index.shape[1]

    grid1 = 10
    h = pl.pallas_call(
        _matmul_body,
        grid=(grid1,),
        in_specs=[
            pl.BlockSpec((n // grid1, d_in), lambda i: (i, 0)),
            pl.BlockSpec((d_in, d_out), lambda i: (0, 0)),
            pl.BlockSpec((n // grid1, 1), lambda i: (i, 0)),
        ],
        out_specs=pl.BlockSpec((n // grid1, d_out), lambda i: (i, 0)),
        out_shape=jax.ShapeDtypeStruct((n, d_out), jnp.float32),
    )(x, W, cj)

    n_tiles = NC * NS
    e_tile = e // n_tiles
    n_chunks = -(-e_tile // CHUNK)
    n_chunks = -(-n_chunks // 6) * 6
    e_pad = n_chunks * CHUNK
    pad = e_pad - e_tile
    n_acc = -(-n // (NS * CHUNK)) * NS * CHUNK
    src = edge_index[0].reshape(n_tiles, e_tile)
    dst = edge_index[1].reshape(n_tiles, e_tile)
    srcp = jnp.concatenate(
        [src, jnp.zeros((n_tiles, pad), jnp.int32)], axis=1)
    dstp = jnp.concatenate(
        [dst, jnp.full((n_tiles, pad), n, jnp.int32)], axis=1
    ).reshape(n_tiles, n_chunks // 2, 2 * CHUNK)

    partials = _make_sc_kernel(n, n_acc, d_out, n_chunks)(h, srcp, dstp)

    grid3 = 5
    rst = pl.pallas_call(
        _combine_body,
        grid=(grid3,),
        in_specs=[
            pl.BlockSpec((NC, n // grid3, d_out), lambda i: (0, i, 0)),
            pl.BlockSpec((n // grid3, 1), lambda i: (i, 0)),
        ],
        out_specs=pl.BlockSpec((n // grid3, d_out), lambda i: (i, 0)),
        out_shape=jax.ShapeDtypeStruct((n, d_out), jnp.float32),
    )(partials, ci)
    return rst


# speedup vs baseline: 1.4906x; 1.4906x over previous
import functools

import jax
import jax.numpy as jnp
from jax import lax
from jax.experimental import pallas as pl
from jax.experimental.pallas import tpu as pltpu
from jax.experimental.pallas import tpu_sc as plsc

NC = 2
NS = 16
CHUNK = 96


def _matmul_body(x_ref, w_ref, cj_ref, o_ref):
    o_ref[...] = (
        jnp.dot(x_ref[...], w_ref[...], preferred_element_type=jnp.float32)
        * cj_ref[...]
    )


def _combine_body(p_ref, ci_ref, o_ref):
    o_ref[...] = (p_ref[0] + p_ref[1]) * ci_ref[...]


def _make_sc_kernel(n, n_acc, d, n_chunks):
    del n
    rows_per_tile = n_acc // NS
    mesh = plsc.VectorSubcoreMesh(
        core_axis_name="c", subcore_axis_name="s",
        num_cores=NC, num_subcores=NS)

    @functools.partial(
        pl.kernel,
        mesh=mesh,
        out_type=jax.ShapeDtypeStruct((NC, n_acc, d), jnp.float32),
        scratch_types=[
            pltpu.VMEM((n_chunks * CHUNK,), jnp.int32),
            pltpu.VMEM((n_chunks, CHUNK), jnp.int32),
            pltpu.VMEM((CHUNK, d), jnp.float32),
            pltpu.VMEM((CHUNK, d), jnp.float32),
            pltpu.VMEM_SHARED((n_acc, d), jnp.float32),
            pltpu.SemaphoreType.DMA,
            pltpu.SemaphoreType.DMA,
            pltpu.SemaphoreType.DMA,
            pltpu.SemaphoreType.DMA,
        ],
    )
    def sc_kernel(h_hbm, src_hbm, dst_hbm, out_hbm,
                  src_v, dst_v, b0, b1, acc,
                  g0, g1, s0, s1):
        c = lax.axis_index("c")
        s = lax.axis_index("s")
        wid = c * NS + s
        bufs = [b0, b1]
        gsems = [g0, g1]
        ssems = [s0, s1]

        def gather(j, b):
            idx = src_v.at[pl.ds(j * CHUNK, CHUNK)]
            return pltpu.make_async_copy(h_hbm.at[idx], bufs[b], gsems[b])

        def scatter(j, b):
            return pltpu.make_async_copy(bufs[b], acc.at[dst_v.at[j]],
                                         ssems[b])

        pltpu.sync_copy(src_hbm.at[wid], src_v)
        pltpu.sync_copy(dst_hbm.at[wid], dst_v)

        zv = jnp.zeros((16,), jnp.float32)

        def zero_row(i, carry):
            for jj in range(d // 16):
                b0[i, pl.ds(jj * 16, 16)] = zv
            return carry

        lax.fori_loop(0, CHUNK, zero_row, 0)
        full, tail = divmod(rows_per_tile, CHUNK)
        for k in range(full):
            pltpu.sync_copy(b0, acc.at[pl.ds(s * rows_per_tile + k * CHUNK, CHUNK)])
        if tail:
            pltpu.sync_copy(
                b0.at[pl.ds(0, tail)],
                acc.at[pl.ds(s * rows_per_tile + full * CHUNK, tail)])
        plsc.subcore_barrier()

        gather(0, 0).start()
        gather(0, 0).wait()
        scatter(0, 0).start(add=True)
        gather(1, 1).start()
        gather(1, 1).wait()
        scatter(1, 1).start(add=True)
        scatter(0, 0).wait()
        gather(2, 0).start()

        @pl.loop(2, n_chunks - 2, step=2)
        def _chunk_pair(j):
            gather(j, 0).wait()
            scatter(j, 0).start(add=True)
            scatter(j - 1, 1).wait()
            gather(j + 1, 1).start()
            gather(j + 1, 1).wait()
            scatter(j + 1, 1).start(add=True)
            scatter(j, 0).wait()
            gather(j + 2, 0).start()

        j_last = n_chunks - 2
        gather(j_last, 0).wait()
        scatter(j_last, 0).start(add=True)
        scatter(j_last - 1, 1).wait()
        gather(j_last + 1, 1).start()
        gather(j_last + 1, 1).wait()
        scatter(j_last + 1, 1).start(add=True)
        scatter(j_last, 0).wait()
        scatter(j_last + 1, 1).wait()
        plsc.subcore_barrier()

        base = s * rows_per_tile
        pltpu.sync_copy(acc.at[pl.ds(base, rows_per_tile)],
                        out_hbm.at[c, pl.ds(base, rows_per_tile)])

    return sc_kernel


def kernel(x, edge_index, cj, ci, W):
    n, d_in = x.shape
    d_out = W.shape[1]
    e = edge_index.shape[1]

    grid1 = 10
    h = pl.pallas_call(
        _matmul_body,
        grid=(grid1,),
        in_specs=[
            pl.BlockSpec((n // grid1, d_in), lambda i: (i, 0)),
            pl.BlockSpec((d_in, d_out), lambda i: (0, 0)),
            pl.BlockSpec((n // grid1, 1), lambda i: (i, 0)),
        ],
        out_specs=pl.BlockSpec((n // grid1, d_out), lambda i: (i, 0)),
        out_shape=jax.ShapeDtypeStruct((n, d_out), jnp.float32),
    )(x, W, cj)

    n_tiles = NC * NS
    e_tile = e // n_tiles
    n_chunks = -(-e_tile // CHUNK)
    n_chunks += n_chunks % 2
    e_pad = n_chunks * CHUNK
    pad = e_pad - e_tile
    n_acc = -(-(n + 1) // (NS * 8)) * NS * 8
    src = edge_index[0].reshape(n_tiles, e_tile)
    dst = edge_index[1].reshape(n_tiles, e_tile)
    srcp = jnp.concatenate(
        [src, jnp.zeros((n_tiles, pad), jnp.int32)], axis=1)
    dstp = jnp.concatenate(
        [dst, jnp.full((n_tiles, pad), n, jnp.int32)], axis=1
    ).reshape(n_tiles, n_chunks, CHUNK)

    partials = _make_sc_kernel(n, n_acc, d_out, n_chunks)(h, srcp, dstp)

    grid3 = 5
    rst = pl.pallas_call(
        _combine_body,
        grid=(grid3,),
        in_specs=[
            pl.BlockSpec((NC, n // grid3, d_out), lambda i: (0, i, 0)),
            pl.BlockSpec((n // grid3, 1), lambda i: (i, 0)),
        ],
        out_specs=pl.BlockSpec((n // grid3, d_out), lambda i: (i, 0)),
        out_shape=jax.ShapeDtypeStruct((n, d_out), jnp.float32),
    )(partials, ci)
    return rst


# speedup vs baseline: 1.7354x; 1.1642x over previous
import functools

import jax
import jax.numpy as jnp
from jax import lax
from jax.experimental import pallas as pl
from jax.experimental.pallas import tpu as pltpu
from jax.experimental.pallas import tpu_sc as plsc

NC = 2
NS = 16
CHUNK = 48


def _matmul_body(x_ref, w_ref, cj_ref, o_ref):
    o_ref[...] = (
        jnp.dot(x_ref[...], w_ref[...], preferred_element_type=jnp.float32)
        * cj_ref[...]
    )


def _combine_body(p_ref, ci_ref, o_ref):
    o_ref[...] = (p_ref[0] + p_ref[1]) * ci_ref[...]


def _make_sc_kernel(n, n_acc, d, n_chunks):
    del n
    rows_per_tile = n_acc // NS
    mesh = plsc.VectorSubcoreMesh(
        core_axis_name="c", subcore_axis_name="s",
        num_cores=NC, num_subcores=NS)

    @functools.partial(
        pl.kernel,
        mesh=mesh,
        out_type=jax.ShapeDtypeStruct((NC, n_acc, d), jnp.float32),
        scratch_types=[
            pltpu.VMEM((n_chunks * CHUNK,), jnp.int32),
            pltpu.VMEM((n_chunks, CHUNK), jnp.int32),
            pltpu.VMEM((CHUNK, d), jnp.float32),
            pltpu.VMEM((CHUNK, d), jnp.float32),
            pltpu.VMEM_SHARED((n_acc, d), jnp.float32),
            pltpu.SemaphoreType.DMA,
            pltpu.SemaphoreType.DMA,
            pltpu.SemaphoreType.DMA,
            pltpu.SemaphoreType.DMA,
        ],
    )
    def sc_kernel(h_hbm, src_hbm, dst_hbm, out_hbm,
                  src_v, dst_v, b0, b1, acc,
                  g0, g1, s0, s1):
        c = lax.axis_index("c")
        s = lax.axis_index("s")
        wid = c * NS + s
        bufs = [b0, b1]
        gsems = [g0, g1]
        ssems = [s0, s1]

        def gather(j, b):
            idx = src_v.at[pl.ds(j * CHUNK, CHUNK)]
            return pltpu.make_async_copy(h_hbm.at[idx], bufs[b], gsems[b])

        def scatter(j, b):
            return pltpu.make_async_copy(bufs[b], acc.at[dst_v.at[j]],
                                         ssems[b])

        pltpu.sync_copy(src_hbm.at[wid], src_v)
        pltpu.sync_copy(dst_hbm.at[wid], dst_v)

        zv = jnp.zeros((16,), jnp.float32)

        def zero_row(i, carry):
            for jj in range(d // 16):
                b0[i, pl.ds(jj * 16, 16)] = zv
            return carry

        lax.fori_loop(0, CHUNK, zero_row, 0)
        full, tail = divmod(rows_per_tile, CHUNK)
        for k in range(full):
            pltpu.sync_copy(b0, acc.at[pl.ds(s * rows_per_tile + k * CHUNK, CHUNK)])
        if tail:
            pltpu.sync_copy(
                b0.at[pl.ds(0, tail)],
                acc.at[pl.ds(s * rows_per_tile + full * CHUNK, tail)])
        plsc.subcore_barrier()

        gather(0, 0).start()
        gather(0, 0).wait()
        scatter(0, 0).start(add=True)
        gather(1, 1).start()
        gather(1, 1).wait()
        scatter(1, 1).start(add=True)
        scatter(0, 0).wait()
        gather(2, 0).start()

        @pl.loop(2, n_chunks - 2, step=2)
        def _chunk_pair(j):
            gather(j, 0).wait()
            scatter(j, 0).start(add=True)
            scatter(j - 1, 1).wait()
            gather(j + 1, 1).start()
            gather(j + 1, 1).wait()
            scatter(j + 1, 1).start(add=True)
            scatter(j, 0).wait()
            gather(j + 2, 0).start()

        j_last = n_chunks - 2
        gather(j_last, 0).wait()
        scatter(j_last, 0).start(add=True)
        scatter(j_last - 1, 1).wait()
        gather(j_last + 1, 1).start()
        gather(j_last + 1, 1).wait()
        scatter(j_last + 1, 1).start(add=True)
        scatter(j_last, 0).wait()
        scatter(j_last + 1, 1).wait()
        plsc.subcore_barrier()

        base = s * rows_per_tile
        pltpu.sync_copy(acc.at[pl.ds(base, rows_per_tile)],
                        out_hbm.at[c, pl.ds(base, rows_per_tile)])

    return sc_kernel


def kernel(x, edge_index, cj, ci, W):
    n, d_in = x.shape
    d_out = W.shape[1]
    e = edge_index.shape[1]

    grid1 = 10
    h = pl.pallas_call(
        _matmul_body,
        grid=(grid1,),
        in_specs=[
            pl.BlockSpec((n // grid1, d_in), lambda i: (i, 0)),
            pl.BlockSpec((d_in, d_out), lambda i: (0, 0)),
            pl.BlockSpec((n // grid1, 1), lambda i: (i, 0)),
        ],
        out_specs=pl.BlockSpec((n // grid1, d_out), lambda i: (i, 0)),
        out_shape=jax.ShapeDtypeStruct((n, d_out), jnp.float32),
    )(x, W, cj)

    n_tiles = NC * NS
    e_tile = e // n_tiles
    n_chunks = -(-e_tile // CHUNK)
    n_chunks += n_chunks % 2
    e_pad = n_chunks * CHUNK
    pad = e_pad - e_tile
    n_acc = -(-(n + 1) // (NS * 8)) * NS * 8
    src = edge_index[0].reshape(n_tiles, e_tile)
    dst = edge_index[1].reshape(n_tiles, e_tile)
    srcp = jnp.concatenate(
        [src, jnp.zeros((n_tiles, pad), jnp.int32)], axis=1)
    dstp = jnp.concatenate(
        [dst, jnp.full((n_tiles, pad), n, jnp.int32)], axis=1
    ).reshape(n_tiles, n_chunks, CHUNK)

    partials = _make_sc_kernel(n, n_acc, d_out, n_chunks)(h, srcp, dstp)

    grid3 = 5
    rst = pl.pallas_call(
        _combine_body,
        grid=(grid3,),
        in_specs=[
            pl.BlockSpec((NC, n // grid3, d_out), lambda i: (0, i, 0)),
            pl.BlockSpec((n // grid3, 1), lambda i: (i, 0)),
        ],
        out_specs=pl.BlockSpec((n // grid3, d_out), lambda i: (i, 0)),
        out_shape=jax.ShapeDtypeStruct((n, d_out), jnp.float32),
    )(partials, ci)
    return rst
